# baseline (speedup 1.0000x reference)
import jax
import jax.numpy as jnp
from jax import lax
from jax.experimental import pallas as pl
from jax.experimental.pallas import tpu as pltpu
from jax.experimental.pallas import tpu_sc as plsc

SEQ_LEN = 2048
D_MODEL = 1024
BATCH = 4

R_SC = 512
R_TC = SEQ_LEN - R_SC
BS = 512

NC = 2
NS = 16
NW = NC * NS
RPW = R_SC // NW
CH = 4
NBUF = 2
NCHUNK = RPW // CH


def _renorm_scale(ss):
    iv = lax.bitcast_convert_type(ss, jnp.int32)
    y = lax.bitcast_convert_type(
        jnp.int32(0x5F3759DF) - lax.shift_right_arithmetic(iv, 1), jnp.float32
    )
    for _ in range(4):
        y = y * (1.5 - 0.5 * ss * y * y)
    nrm = jnp.full((16,), ss * y)
    return jnp.where(
        nrm > 1.0, jnp.full((16,), 1.0) / (nrm + 1e-7), jnp.full((16,), 1.0)
    )


def _sc_body(x_hbm, t_hbm, o_hbm, *scr):
    xbufs = scr[0:NBUF]
    tbufs = scr[NBUF : 2 * NBUF]
    sin = scr[2 * NBUF : 3 * NBUF]
    sout = scr[3 * NBUF : 4 * NBUF]
    wid = lax.axis_index("s") * NC + lax.axis_index("c")
    base = R_TC + wid * RPW

    def start_in(g):
        s = g % NBUF
        row = base + g * CH
        pltpu.make_async_copy(x_hbm.at[pl.ds(row, CH)], xbufs[s], sin[s]).start()
        pltpu.make_async_copy(t_hbm.at[pl.ds(row, CH)], tbufs[s], sin[s]).start()

    def wait_in(g):
        s = g % NBUF
        pltpu.make_async_copy(x_hbm.at[pl.ds(0, CH)], xbufs[s], sin[s]).wait()
        pltpu.make_async_copy(t_hbm.at[pl.ds(0, CH)], tbufs[s], sin[s]).wait()

    def start_out(g):
        s = g % NBUF
        row = base - R_TC + g * CH
        pltpu.make_async_copy(xbufs[s], o_hbm.at[pl.ds(row, CH)], sout[s]).start()

    def wait_out(g):
        s = g % NBUF
        pltpu.make_async_copy(xbufs[s], o_hbm.at[pl.ds(0, CH)], sout[s]).wait()

    for g in range(min(NBUF - 1, NCHUNK)):
        start_in(g)

    for g in range(NCHUNK):
        s = g % NBUF
        wait_in(g)
        xb, tb = xbufs[s], tbufs[s]

        def row_body(r, c, xb=xb, tb=tb):

            @plsc.parallel_loop(
                0, D_MODEL, step=16, unroll=8, carry=jnp.zeros((16,), jnp.float32)
            )
            def ss_acc(i, acc):
                v = tb[r, pl.ds(i, 16)]
                return acc + v * v

            ss = ss_acc[0]
            for i in range(1, 16):
                ss = ss + ss_acc[i]
            scale = _renorm_scale(ss)

            @plsc.parallel_loop(0, D_MODEL, step=16, unroll=8)
            def _add(i):
                sl = pl.ds(i, 16)
                e = tb[r, sl] * scale
                xb[r, 0, sl] += e
                xb[r, 1, sl] += e
                xb[r, 2, sl] += e
                xb[r, 3, sl] += e

            return c

        lax.fori_loop(0, CH, row_body, 0)
        start_out(g)
        if g >= 1:
            wait_out(g - 1)
        h = g + NBUF - 1
        if h < NCHUNK:
            start_in(h)
    wait_out(NCHUNK - 1)


def _sc_add(x, table):
    mesh = plsc.VectorSubcoreMesh(
        core_axis_name="c", subcore_axis_name="s", num_cores=NC, num_subcores=NS
    )
    scratch = (
        [pltpu.VMEM((CH, BATCH, D_MODEL), jnp.float32)] * NBUF
        + [pltpu.VMEM((CH, D_MODEL), jnp.float32)] * NBUF
        + [pltpu.SemaphoreType.DMA] * (2 * NBUF)
    )
    return pl.kernel(
        _sc_body,
        out_type=jax.ShapeDtypeStruct((R_SC, BATCH, D_MODEL), jnp.float32),
        mesh=mesh,
        scratch_types=scratch,
    )(x, table)


def _tc_kern(x_ref, t_ref, o_ref):
    t = t_ref[...]
    norm = jnp.sqrt(jnp.sum(t * t, axis=1, keepdims=True))
    scale = jnp.where(norm > 1.0, 1.0 / (norm + 1e-7), 1.0)
    emb = t * scale
    for b in range(BATCH):
        o_ref[:, b, :] = x_ref[:, b, :] + emb


def _tc_add(x, table):
    return pl.pallas_call(
        _tc_kern,
        grid=(R_TC // BS,),
        in_specs=[
            pl.BlockSpec((BS, BATCH, D_MODEL), lambda i: (i, 0, 0)),
            pl.BlockSpec((BS, D_MODEL), lambda i: (i, 0)),
        ],
        out_specs=pl.BlockSpec((BS, BATCH, D_MODEL), lambda i: (i, 0, 0)),
        out_shape=jax.ShapeDtypeStruct((SEQ_LEN, BATCH, D_MODEL), x.dtype),
        compiler_params=pltpu.CompilerParams(
            dimension_semantics=("parallel",),
        ),
    )(x, table)


def _merge_kern(tc_ref, sc_ref, o_ref):
    o_ref[...] = sc_ref[...]


def _merge(tc_out, sc_rows):
    return pl.pallas_call(
        _merge_kern,
        grid=(R_SC // BS,),
        in_specs=[
            pl.BlockSpec(memory_space=pl.ANY),
            pl.BlockSpec((BS, BATCH, D_MODEL), lambda i: (i, 0, 0)),
        ],
        out_specs=pl.BlockSpec(
            (BS, BATCH, D_MODEL), lambda i: (i + R_TC // BS, 0, 0)
        ),
        out_shape=jax.ShapeDtypeStruct((SEQ_LEN, BATCH, D_MODEL), jnp.float32),
        input_output_aliases={0: 0},
    )(tc_out, sc_rows)


def kernel(x, table):
    sc_rows = _sc_add(x, table)
    tc_out = _tc_add(x, table)
    return _merge(tc_out, sc_rows)


# speedup vs baseline: 1.9054x; 1.9054x over previous
import jax
import jax.numpy as jnp
from jax.experimental import pallas as pl
from jax.experimental.pallas import tpu as pltpu

SEQ_LEN = 2048
D_MODEL = 1024
BATCH = 4
BS = 512


def _kern(x_ref, t_ref, o_ref):
    t = t_ref[...]
    norm = jnp.sqrt(jnp.sum(t * t, axis=1, keepdims=True))
    scale = jnp.where(norm > 1.0, 1.0 / (norm + 1e-7), 1.0)
    emb = t * scale
    for b in range(BATCH):
        o_ref[:, b, :] = x_ref[:, b, :] + emb


def kernel(x, table):
    return pl.pallas_call(
        _kern,
        grid=(SEQ_LEN // BS,),
        in_specs=[
            pl.BlockSpec((BS, BATCH, D_MODEL), lambda i: (i, 0, 0)),
            pl.BlockSpec((BS, D_MODEL), lambda i: (i, 0)),
        ],
        out_specs=pl.BlockSpec((BS, BATCH, D_MODEL), lambda i: (i, 0, 0)),
        out_shape=jax.ShapeDtypeStruct((SEQ_LEN, BATCH, D_MODEL), x.dtype),
        compiler_params=pltpu.CompilerParams(
            dimension_semantics=("parallel",),
        ),
    )(x, table)
